# baseline (speedup 1.0000x reference)
import functools

import jax
import jax.numpy as jnp
from jax.experimental import pallas as pl


def _vq_body(f_ref, cb_ref, fn_ref, cbn_ref, idx_ref, q_ref, loss_ref, *,
             blk, K):
    f = f_ref[...]
    cb = cb_ref[...]
    fn = fn_ref[...]
    cbn = cbn_ref[...]
    scores = jax.lax.dot_general(
        f, cb, dimension_numbers=(((1,), (1,)), ((), ())),
        preferred_element_type=jnp.float32)
    d = fn + cbn - 2.0 * scores
    dmin = jnp.min(d, axis=1, keepdims=True)
    kiota = jax.lax.broadcasted_iota(jnp.int32, (blk, K), 1)
    idx = jnp.min(jnp.where(d == dmin, kiota, K), axis=1)
    idx_ref[...] = idx[:, None].astype(jnp.int32)
    onehot = (kiota == idx[:, None]).astype(jnp.float32)
    q_ref[...] = jax.lax.dot_general(
        onehot, cb, dimension_numbers=(((1,), (0,)), ((), ())),
        preferred_element_type=jnp.float32)
    part = jnp.sum(dmin, axis=0, keepdims=True)

    @pl.when(pl.program_id(0) == 0)
    def _init():
        loss_ref[...] = jnp.zeros_like(part)

    loss_ref[...] += part


def _vq(flat, codebook, *, blk=256):
    n, d = flat.shape
    k = codebook.shape[0]
    grid = n // blk
    fn = jnp.sum(flat ** 2, axis=1, keepdims=True)
    cbn = jnp.sum(codebook ** 2, axis=1)[None, :]
    idx, q, loss = pl.pallas_call(
        functools.partial(_vq_body, blk=blk, K=k),
        grid=(grid,),
        in_specs=[
            pl.BlockSpec((blk, d), lambda i: (i, 0)),
            pl.BlockSpec((k, d), lambda i: (0, 0)),
            pl.BlockSpec((blk, 1), lambda i: (i, 0)),
            pl.BlockSpec((1, k), lambda i: (0, 0)),
        ],
        out_specs=[
            pl.BlockSpec((blk, 1), lambda i: (i, 0)),
            pl.BlockSpec((blk, d), lambda i: (i, 0)),
            pl.BlockSpec((1, 1), lambda i: (0, 0)),
        ],
        out_shape=[
            jax.ShapeDtypeStruct((n, 1), jnp.int32),
            jax.ShapeDtypeStruct((n, d), jnp.float32),
            jax.ShapeDtypeStruct((1, 1), jnp.float32),
        ],
    )(flat, codebook, fn, cbn)
    return idx, q, loss[0, 0]


_DNL = ('NHWC', 'HWIO', 'NHWC')


def _conv2d_nhwc(x, w, b, stride, pad):
    y = jax.lax.conv_general_dilated(
        x, w.transpose(2, 3, 1, 0), (stride, stride),
        [(pad, pad), (pad, pad)], dimension_numbers=_DNL)
    return y + b[None, None, None, :]


def _up8(n):
    return (n + 7) // 8 * 8


def _convt_body(x_ref, w_ref, b_ref, out_ref, *, mp, shifts, act):
    xv = x_ref[0]
    b = b_ref[...]
    lhs_cache = {}
    ys = []
    for p in range(4):
        key = shifts[p]
        if key not in lhs_cache:
            lhs_cache[key] = jnp.concatenate(
                [xv[s:s + mp] for s in key], axis=1)
        ys.append(jax.lax.dot_general(
            lhs_cache[key], w_ref[p],
            dimension_numbers=(((1,), (0,)), ((), ())),
            preferred_element_type=jnp.float32))
    y = jnp.concatenate(ys, axis=1) + b
    out_ref[0] = act(y)


def _convt_phases(xg, wp, bias, shifts, act):
    B, Hp, Wp, Ci = xg.shape
    Co = wp.shape[2]
    mp = _up8(Hp * Wp)
    tail = _up8(2 * Wp + 8)
    xf = jnp.zeros((B, mp + tail, Ci), jnp.bfloat16)
    xf = jax.lax.dynamic_update_slice(
        xf, xg.reshape(B, Hp * Wp, Ci).astype(jnp.bfloat16), (0, 0, 0))
    b4 = jnp.tile(bias.astype(jnp.float32), 4).reshape(1, 4 * Co)
    out = pl.pallas_call(
        functools.partial(_convt_body, mp=mp, shifts=shifts, act=act),
        grid=(B,),
        in_specs=[
            pl.BlockSpec((1, mp + tail, Ci), lambda i: (i, 0, 0)),
            pl.BlockSpec((4, 4 * Ci, Co), lambda i: (0, 0, 0)),
            pl.BlockSpec((1, 4 * Co), lambda i: (0, 0)),
        ],
        out_specs=pl.BlockSpec((1, mp, 4 * Co), lambda i: (i, 0, 0)),
        out_shape=jax.ShapeDtypeStruct((B, mp, 4 * Co), jnp.float32),
    )(xf, wp, b4)
    return out


_TAPS = ((0, 0), (0, 1), (1, 0), (1, 1))


def _interleave(y, B, Hp, Wp, Co, crop_h, crop_w):
    y = y[:, :Hp * Wp, :].reshape(B, Hp, Wp, 2, 2, Co)
    y = y.transpose(0, 1, 3, 2, 4, 5)
    y = y.reshape(B, 2 * Hp, 2 * Wp, Co)
    return y[:, :crop_h, :crop_w, :]


def _convt_s2k4p1(x, w, b, act):
    B, H, W, Ci = x.shape
    Co = w.shape[1]
    xg = jnp.pad(x, ((0, 0), (1, 1), (1, 1), (0, 0)))
    Wp = W + 2
    wp = jnp.stack([
        jnp.concatenate([w[:, :, 3 - 2 * a - r, 3 - 2 * bb - c]
                         for (a, bb) in _TAPS], axis=0)
        for r in (0, 1) for c in (0, 1)]).astype(jnp.bfloat16)
    shifts = [tuple((r + a) * Wp + (c + bb) for (a, bb) in _TAPS)
              for r in (0, 1) for c in (0, 1)]
    y = _convt_phases(xg, wp, b, shifts, act)
    return _interleave(y, B, H + 2, Wp, Co, 2 * H, 2 * W)


def _convt_s2k4p0op1(x, w, b, act):
    B, H, W, Ci = x.shape
    Co = w.shape[1]
    xg = jnp.pad(x, ((0, 0), (1, 2), (1, 2), (0, 0)))
    Hp, Wp = H + 3, W + 3
    wp = jnp.stack([
        jnp.concatenate([w[:, :, r + 2 * a, c + 2 * bb]
                         for (a, bb) in _TAPS], axis=0)
        for r in (0, 1) for c in (0, 1)]).astype(jnp.bfloat16)
    shifts = [tuple((1 - a) * Wp + (1 - bb) for (a, bb) in _TAPS)
              for r in (0, 1) for c in (0, 1)]
    y = _convt_phases(xg, wp, b, shifts, act)
    return _interleave(y, B, Hp, Wp, Co, 2 * H + 3, 2 * W + 3)


def kernel(x, ew1, eb1, ew2, eb2, ew3, eb3, codebook, dw1, db1, dw2, db2,
           dw3, db3):
    commitment_cost = 0.25
    xl = x.transpose(0, 2, 3, 1)
    z = jax.nn.relu(_conv2d_nhwc(xl, ew1, eb1, 2, 1))
    z = jax.nn.relu(_conv2d_nhwc(z, ew2, eb2, 2, 1))
    z = _conv2d_nhwc(z, ew3, eb3, 2, 1)
    B, H, W, D = z.shape
    flat = z.reshape(-1, D)

    idx, quantized, loss_sum = _vq(flat, codebook)
    vq_loss = (1.0 + commitment_cost) * loss_sum / (flat.shape[0] * D)

    z_q = quantized.reshape(B, H, W, D)
    h = _convt_s2k4p1(z_q, dw1, db1, jax.nn.relu)
    h = _convt_s2k4p1(h, dw2, db2, jax.nn.relu)
    x_recon = _convt_s2k4p0op1(h, dw3, db3, jax.nn.sigmoid)
    return (x_recon.transpose(0, 3, 1, 2), vq_loss, idx)


# speedup vs baseline: 8.7602x; 8.7602x over previous
import functools

import jax
import jax.numpy as jnp
from jax.experimental import pallas as pl


def _vq_body(f_ref, cb_ref, fn_ref, cbn_ref, idx_ref, q_ref, loss_ref, *,
             blk, K):
    f = f_ref[...]
    cb = cb_ref[...]
    fn = fn_ref[...]
    cbn = cbn_ref[...]
    scores = jax.lax.dot_general(
        f, cb, dimension_numbers=(((1,), (1,)), ((), ())),
        preferred_element_type=jnp.float32)
    d = fn + cbn - 2.0 * scores
    dmin = jnp.min(d, axis=1, keepdims=True)
    kiota = jax.lax.broadcasted_iota(jnp.int32, (blk, K), 1)
    idx = jnp.min(jnp.where(d == dmin, kiota, K), axis=1)
    idx_ref[...] = idx[:, None].astype(jnp.int32)
    onehot = (kiota == idx[:, None]).astype(jnp.float32)
    q_ref[...] = jax.lax.dot_general(
        onehot, cb, dimension_numbers=(((1,), (0,)), ((), ())),
        preferred_element_type=jnp.float32)
    part = jnp.sum(dmin, axis=0, keepdims=True)

    @pl.when(pl.program_id(0) == 0)
    def _init():
        loss_ref[...] = jnp.zeros_like(part)

    loss_ref[...] += part


def _vq(flat, codebook, *, blk=256):
    n, d = flat.shape
    k = codebook.shape[0]
    grid = n // blk
    fn = jnp.sum(flat ** 2, axis=1, keepdims=True)
    cbn = jnp.sum(codebook ** 2, axis=1)[None, :]
    idx, q, loss = pl.pallas_call(
        functools.partial(_vq_body, blk=blk, K=k),
        grid=(grid,),
        in_specs=[
            pl.BlockSpec((blk, d), lambda i: (i, 0)),
            pl.BlockSpec((k, d), lambda i: (0, 0)),
            pl.BlockSpec((blk, 1), lambda i: (i, 0)),
            pl.BlockSpec((1, k), lambda i: (0, 0)),
        ],
        out_specs=[
            pl.BlockSpec((blk, 1), lambda i: (i, 0)),
            pl.BlockSpec((blk, d), lambda i: (i, 0)),
            pl.BlockSpec((1, 1), lambda i: (0, 0)),
        ],
        out_shape=[
            jax.ShapeDtypeStruct((n, 1), jnp.int32),
            jax.ShapeDtypeStruct((n, d), jnp.float32),
            jax.ShapeDtypeStruct((1, 1), jnp.float32),
        ],
    )(flat, codebook, fn, cbn)
    return idx, q, loss[0, 0]


_DNL = ('NHWC', 'HWIO', 'NHWC')


def _conv2d_nhwc(x, w, b, stride, pad):
    y = jax.lax.conv_general_dilated(
        x, w.transpose(2, 3, 1, 0), (stride, stride),
        [(pad, pad), (pad, pad)], dimension_numbers=_DNL)
    return y + b[None, None, None, :]


_TAPS = ((0, 0), (0, 1), (1, 0), (1, 1))


def _dec_body(x_ref, w_ref, b_ref, out_ref, *, gh, gw, valid, shifts,
              store_rows, store_cols, relu_out):
    r_in = x_ref.shape[1]
    ci = x_ref.shape[3]
    xv = x_ref[0].reshape(r_in * gw, ci).astype(jnp.bfloat16)
    m = gh * gw
    b = b_ref[...]
    out_ref[...] = jnp.zeros_like(out_ref)
    for p, (pr, pc) in enumerate(((0, 0), (0, 1), (1, 0), (1, 1))):
        lhs = jnp.concatenate(
            [xv[s:s + m] for s in shifts[p]], axis=1)
        acc = jax.lax.dot_general(
            lhs, w_ref[p], dimension_numbers=(((1,), (0,)), ((), ())),
            preferred_element_type=jnp.float32)
        y = acc + b
        y = jnp.maximum(y, 0.0) if relu_out else jax.nn.sigmoid(y)
        y = y.astype(out_ref.dtype).reshape(gh, gw, -1)[:valid, :valid, :]
        out_ref[0, pl.Slice(store_rows + pr, valid, 2),
                pl.Slice(store_cols + pc, valid, 2), :] = y


def _dec_layer(xbuf, w, bias, *, gh, gw, valid, taps_kh, shifts,
               out_rows, out_cols, out_co, store_rows, store_cols,
               relu_out):
    B = xbuf.shape[0]
    Ci = xbuf.shape[3]
    Co = w.shape[1]
    wp = jnp.stack([
        jnp.concatenate([w[:, :, kh, kw] for (kh, kw) in taps_kh[p]], axis=0)
        for p in range(4)]).astype(jnp.bfloat16)
    out = pl.pallas_call(
        functools.partial(
            _dec_body, gh=gh, gw=gw, valid=valid, shifts=shifts,
            store_rows=store_rows, store_cols=store_cols, relu_out=relu_out),
        grid=(B,),
        in_specs=[
            pl.BlockSpec((1,) + xbuf.shape[1:], lambda i: (i, 0, 0, 0)),
            pl.BlockSpec((4, 4 * Ci, Co), lambda i: (0, 0, 0)),
            pl.BlockSpec((1, Co), lambda i: (0, 0)),
        ],
        out_specs=pl.BlockSpec((1, out_rows, out_cols, Co),
                               lambda i: (i, 0, 0, 0)),
        out_shape=jax.ShapeDtypeStruct((B, out_rows, out_cols, Co),
                                       jnp.float32),
    )(xbuf, wp, bias.reshape(1, Co).astype(jnp.float32))
    return out


def _dec3_body(x_ref, w_ref, b_ref, out_ref, *, gh, gw, shifts):
    r_in = x_ref.shape[1]
    ci = x_ref.shape[3]
    xv = x_ref[0].reshape(r_in * gw, ci).astype(jnp.bfloat16)
    m = gh * gw
    lhs = jnp.concatenate([xv[s:s + m] for s in shifts], axis=1)
    acc = jax.lax.dot_general(
        lhs, w_ref[...], dimension_numbers=(((1,), (0,)), ((), ())),
        preferred_element_type=jnp.float32)
    y = jax.nn.sigmoid(acc + b_ref[...])
    out_ref[0] = y.astype(jnp.bfloat16)


def kernel(x, ew1, eb1, ew2, eb2, ew3, eb3, codebook, dw1, db1, dw2, db2,
           dw3, db3):
    commitment_cost = 0.25
    xl = x.transpose(0, 2, 3, 1)
    z = jax.nn.relu(_conv2d_nhwc(xl, ew1, eb1, 2, 1))
    z = jax.nn.relu(_conv2d_nhwc(z, ew2, eb2, 2, 1))
    z = _conv2d_nhwc(z, ew3, eb3, 2, 1)
    B, H, W, D = z.shape
    flat = z.reshape(-1, D)

    idx, quantized, loss_sum = _vq(flat, codebook)
    vq_loss = (1.0 + commitment_cost) * loss_sum / (flat.shape[0] * D)

    zq = jnp.pad(quantized.reshape(B, H, W, D),
                 ((0, 0), (1, 5), (1, 3), (0, 0)))
    taps1 = [[(3 - 2 * a - r, 3 - 2 * bb - c) for (a, bb) in _TAPS]
             for r in (0, 1) for c in (0, 1)]
    shifts1 = [tuple((r + a) * 32 + (c + bb) for (a, bb) in _TAPS)
               for r in (0, 1) for c in (0, 1)]
    h1 = _dec_layer(zq, dw1, db1, gh=30, gw=32, valid=28, taps_kh=taps1,
                    shifts=shifts1, out_rows=64, out_cols=64, out_co=64,
                    store_rows=1, store_cols=1, relu_out=True)
    shifts2 = [tuple((r + a) * 64 + (c + bb) for (a, bb) in _TAPS)
               for r in (0, 1) for c in (0, 1)]
    h2 = _dec_layer(h1, dw2, db2, gh=58, gw=64, valid=56, taps_kh=taps1,
                    shifts=shifts2, out_rows=120, out_cols=120, out_co=32,
                    store_rows=1, store_cols=1, relu_out=True)
    Ci3, Co3 = dw3.shape[0], dw3.shape[1]
    w3 = jnp.concatenate([
        jnp.concatenate([dw3[:, :, r + 2 * a, c + 2 * bb]
                         for r in (0, 1) for c in (0, 1)], axis=1)
        for (a, bb) in _TAPS], axis=0).astype(jnp.bfloat16)
    b3 = jnp.tile(db3, 4).reshape(1, 4 * Co3).astype(jnp.float32)
    shifts3 = tuple((1 - a) * 120 + (1 - bb) for (a, bb) in _TAPS)
    m3 = 115 * 120
    y3 = pl.pallas_call(
        functools.partial(_dec3_body, gh=115, gw=120, shifts=shifts3),
        grid=(B,),
        in_specs=[
            pl.BlockSpec((1, 120, 120, Ci3), lambda i: (i, 0, 0, 0)),
            pl.BlockSpec((4 * Ci3, 4 * Co3), lambda i: (0, 0)),
            pl.BlockSpec((1, 4 * Co3), lambda i: (0, 0)),
        ],
        out_specs=pl.BlockSpec((1, m3, 4 * Co3), lambda i: (i, 0, 0)),
        out_shape=jax.ShapeDtypeStruct((B, m3, 4 * Co3), jnp.bfloat16),
    )(h2, w3, b3)
    y3 = y3.reshape(B, 115, 120, 2, 2, Co3)[:, :114, :114]
    y3 = y3.transpose(0, 1, 3, 2, 4, 5).reshape(B, 228, 228, Co3)
    x_recon = y3[:, :227, :227, :].astype(jnp.float32)
    return (x_recon.transpose(0, 3, 1, 2), vq_loss, idx)


# speedup vs baseline: 10.0526x; 1.1475x over previous
import functools

import jax
import jax.numpy as jnp
from jax.experimental import pallas as pl


def _vq_body(f_ref, cb_ref, fn_ref, cbn_ref, idx_ref, q_ref, loss_ref, *,
             blk, K):
    f = f_ref[...]
    cb = cb_ref[...]
    fn = fn_ref[...]
    cbn = cbn_ref[...]
    scores = jax.lax.dot_general(
        f, cb, dimension_numbers=(((1,), (1,)), ((), ())),
        preferred_element_type=jnp.float32)
    d = fn + cbn - 2.0 * scores
    dmin = jnp.min(d, axis=1, keepdims=True)
    kiota = jax.lax.broadcasted_iota(jnp.int32, (blk, K), 1)
    idx = jnp.min(jnp.where(d == dmin, kiota, K), axis=1)
    idx_ref[...] = idx[:, None].astype(jnp.int32)
    onehot = (kiota == idx[:, None]).astype(jnp.float32)
    q_ref[...] = jax.lax.dot_general(
        onehot, cb, dimension_numbers=(((1,), (0,)), ((), ())),
        preferred_element_type=jnp.float32)
    part = jnp.sum(dmin, axis=0, keepdims=True)

    @pl.when(pl.program_id(0) == 0)
    def _init():
        loss_ref[...] = jnp.zeros_like(part)

    loss_ref[...] += part


def _vq(flat, codebook, *, blk=256):
    n, d = flat.shape
    k = codebook.shape[0]
    grid = n // blk
    fn = jnp.sum(flat ** 2, axis=1, keepdims=True)
    cbn = jnp.sum(codebook ** 2, axis=1)[None, :]
    idx, q, loss = pl.pallas_call(
        functools.partial(_vq_body, blk=blk, K=k),
        grid=(grid,),
        in_specs=[
            pl.BlockSpec((blk, d), lambda i: (i, 0)),
            pl.BlockSpec((k, d), lambda i: (0, 0)),
            pl.BlockSpec((blk, 1), lambda i: (i, 0)),
            pl.BlockSpec((1, k), lambda i: (0, 0)),
        ],
        out_specs=[
            pl.BlockSpec((blk, 1), lambda i: (i, 0)),
            pl.BlockSpec((blk, d), lambda i: (i, 0)),
            pl.BlockSpec((1, 1), lambda i: (0, 0)),
        ],
        out_shape=[
            jax.ShapeDtypeStruct((n, 1), jnp.int32),
            jax.ShapeDtypeStruct((n, d), jnp.float32),
            jax.ShapeDtypeStruct((1, 1), jnp.float32),
        ],
    )(flat, codebook, fn, cbn)
    return idx, q, loss[0, 0]


_DNL = ('NHWC', 'HWIO', 'NHWC')


def _conv2d_nhwc(x, w, b, stride, pad):
    y = jax.lax.conv_general_dilated(
        x, w.transpose(2, 3, 1, 0), (stride, stride),
        [(pad, pad), (pad, pad)], dimension_numbers=_DNL)
    return y + b[None, None, None, :]


_TAPS = ((0, 0), (0, 1), (1, 0), (1, 1))


def _dec_body(x_ref, w_ref, b_ref, out_ref, *, gh, gw, valid, shifts,
              store_rows, store_cols, relu_out):
    r_in = x_ref.shape[1]
    ci = x_ref.shape[3]
    xv = x_ref[0].reshape(r_in * gw, ci).astype(jnp.bfloat16)
    m = gh * gw
    b = b_ref[...]
    out_ref[...] = jnp.zeros_like(out_ref)
    for p, (pr, pc) in enumerate(((0, 0), (0, 1), (1, 0), (1, 1))):
        lhs = jnp.concatenate(
            [xv[s:s + m] for s in shifts[p]], axis=1)
        acc = jax.lax.dot_general(
            lhs, w_ref[p], dimension_numbers=(((1,), (0,)), ((), ())),
            preferred_element_type=jnp.float32)
        y = acc + b
        y = jnp.maximum(y, 0.0) if relu_out else jax.nn.sigmoid(y)
        y = y.astype(out_ref.dtype).reshape(gh, gw, -1)[:valid, :valid, :]
        out_ref[0, pl.Slice(store_rows + pr, valid, 2),
                pl.Slice(store_cols + pc, valid, 2), :] = y


def _dec_layer(xbuf, w, bias, *, gh, gw, valid, taps_kh, shifts,
               out_rows, out_cols, out_co, store_rows, store_cols,
               relu_out):
    B = xbuf.shape[0]
    Ci = xbuf.shape[3]
    Co = w.shape[1]
    wp = jnp.stack([
        jnp.concatenate([w[:, :, kh, kw] for (kh, kw) in taps_kh[p]], axis=0)
        for p in range(4)]).astype(jnp.bfloat16)
    out = pl.pallas_call(
        functools.partial(
            _dec_body, gh=gh, gw=gw, valid=valid, shifts=shifts,
            store_rows=store_rows, store_cols=store_cols, relu_out=relu_out),
        grid=(B,),
        in_specs=[
            pl.BlockSpec((1,) + xbuf.shape[1:], lambda i: (i, 0, 0, 0)),
            pl.BlockSpec((4, 4 * Ci, Co), lambda i: (0, 0, 0)),
            pl.BlockSpec((1, Co), lambda i: (0, 0)),
        ],
        out_specs=pl.BlockSpec((1, out_rows, out_cols, Co),
                               lambda i: (i, 0, 0, 0)),
        out_shape=jax.ShapeDtypeStruct((B, out_rows, out_cols, Co),
                                       jnp.float32),
    )(xbuf, wp, bias.reshape(1, Co).astype(jnp.float32))
    return out


def _dec3_body(x_ref, w_ref, b_ref, out_ref, *, gh, gw, shifts, co_out, hout):
    r_in = x_ref.shape[1]
    ci = x_ref.shape[3]
    xv = x_ref[0].reshape(r_in * gw, ci).astype(jnp.bfloat16)
    m = gh * gw
    lhs = jnp.concatenate([xv[s:s + m] for s in shifts], axis=1)
    acc = jax.lax.dot_general(
        lhs, w_ref[...], dimension_numbers=(((1,), (0,)), ((), ())),
        preferred_element_type=jnp.float32)
    y = jax.nn.sigmoid(acc + b_ref[...]).reshape(gh, gw, 4 * co_out)
    half = (hout + 1) // 2
    for co in range(co_out):
        for r in range(2):
            nr = half if r == 0 else hout - half
            t = y[:nr, :half, 2 * (2 * co + r):2 * (2 * co + r) + 2]
            t = t.reshape(nr, 2 * half)
            wpad = 2 * out_ref.shape[4] - 2 * half
            t = jnp.pad(t, ((0, 0), (0, wpad)))
            out_ref[0, co, pl.Slice(r, nr, 2), :, :] = (
                t.reshape(nr, 2, out_ref.shape[4]))


def kernel(x, ew1, eb1, ew2, eb2, ew3, eb3, codebook, dw1, db1, dw2, db2,
           dw3, db3):
    commitment_cost = 0.25
    xl = x.transpose(0, 2, 3, 1)
    z = jax.nn.relu(_conv2d_nhwc(xl, ew1, eb1, 2, 1))
    z = jax.nn.relu(_conv2d_nhwc(z, ew2, eb2, 2, 1))
    z = _conv2d_nhwc(z, ew3, eb3, 2, 1)
    B, H, W, D = z.shape
    flat = z.reshape(-1, D)

    idx, quantized, loss_sum = _vq(flat, codebook)
    vq_loss = (1.0 + commitment_cost) * loss_sum / (flat.shape[0] * D)

    zq = jnp.pad(quantized.reshape(B, H, W, D),
                 ((0, 0), (1, 5), (1, 3), (0, 0)))
    taps1 = [[(3 - 2 * a - r, 3 - 2 * bb - c) for (a, bb) in _TAPS]
             for r in (0, 1) for c in (0, 1)]
    shifts1 = [tuple((r + a) * 32 + (c + bb) for (a, bb) in _TAPS)
               for r in (0, 1) for c in (0, 1)]
    h1 = _dec_layer(zq, dw1, db1, gh=30, gw=32, valid=28, taps_kh=taps1,
                    shifts=shifts1, out_rows=64, out_cols=64, out_co=64,
                    store_rows=1, store_cols=1, relu_out=True)
    shifts2 = [tuple((r + a) * 64 + (c + bb) for (a, bb) in _TAPS)
               for r in (0, 1) for c in (0, 1)]
    h2 = _dec_layer(h1, dw2, db2, gh=58, gw=64, valid=56, taps_kh=taps1,
                    shifts=shifts2, out_rows=120, out_cols=120, out_co=32,
                    store_rows=1, store_cols=1, relu_out=True)
    Ci3, Co3 = dw3.shape[0], dw3.shape[1]
    w3 = jnp.concatenate([
        jnp.stack([dw3[:, co, r + 2 * a, c + 2 * bb]
                   for co in range(Co3) for r in (0, 1) for c in (0, 1)],
                  axis=1)
        for (a, bb) in _TAPS], axis=0).astype(jnp.bfloat16)
    b3 = jnp.repeat(db3, 4).reshape(1, 4 * Co3).astype(jnp.float32)
    shifts3 = tuple((1 - a) * 120 + (1 - bb) for (a, bb) in _TAPS)
    hout = 2 * 112 + 3
    x_recon = pl.pallas_call(
        functools.partial(_dec3_body, gh=115, gw=120, shifts=shifts3,
                          co_out=Co3, hout=hout),
        grid=(B,),
        in_specs=[
            pl.BlockSpec((1, 120, 120, Ci3), lambda i: (i, 0, 0, 0)),
            pl.BlockSpec((4 * Ci3, 4 * Co3), lambda i: (0, 0)),
            pl.BlockSpec((1, 4 * Co3), lambda i: (0, 0)),
        ],
        out_specs=pl.BlockSpec((1, Co3, hout, 2, 128),
                               lambda i: (i, 0, 0, 0, 0)),
        out_shape=jax.ShapeDtypeStruct((B, Co3, hout, 2, 128), jnp.float32),
    )(h2, w3, b3)
    x_recon = x_recon.reshape(B, Co3, hout, 256)[:, :, :, :hout]
    return (x_recon, vq_loss, idx)


# speedup vs baseline: 10.0882x; 1.0035x over previous
import functools

import jax
import jax.numpy as jnp
from jax.experimental import pallas as pl


def _vq_body(f_ref, cb_ref, fn_ref, cbn_ref, idx_ref, q_ref, loss_ref, *,
             blk, K):
    f = f_ref[...]
    cb = cb_ref[...]
    fn = fn_ref[...]
    cbn = cbn_ref[...]
    scores = jax.lax.dot_general(
        f, cb, dimension_numbers=(((1,), (1,)), ((), ())),
        preferred_element_type=jnp.float32)
    d = fn + cbn - 2.0 * scores
    dmin = jnp.min(d, axis=1, keepdims=True)
    kiota = jax.lax.broadcasted_iota(jnp.int32, (blk, K), 1)
    idx = jnp.min(jnp.where(d == dmin, kiota, K), axis=1)
    idx_ref[...] = idx[:, None].astype(jnp.int32)
    onehot = (kiota == idx[:, None]).astype(jnp.float32)
    q_ref[...] = jax.lax.dot_general(
        onehot, cb, dimension_numbers=(((1,), (0,)), ((), ())),
        preferred_element_type=jnp.float32)
    part = jnp.sum(dmin, axis=0, keepdims=True)

    @pl.when(pl.program_id(0) == 0)
    def _init():
        loss_ref[...] = jnp.zeros_like(part)

    loss_ref[...] += part


def _vq(flat, codebook, *, blk=256):
    n, d = flat.shape
    k = codebook.shape[0]
    grid = n // blk
    fn = jnp.sum(flat ** 2, axis=1, keepdims=True)
    cbn = jnp.sum(codebook ** 2, axis=1)[None, :]
    idx, q, loss = pl.pallas_call(
        functools.partial(_vq_body, blk=blk, K=k),
        grid=(grid,),
        in_specs=[
            pl.BlockSpec((blk, d), lambda i: (i, 0)),
            pl.BlockSpec((k, d), lambda i: (0, 0)),
            pl.BlockSpec((blk, 1), lambda i: (i, 0)),
            pl.BlockSpec((1, k), lambda i: (0, 0)),
        ],
        out_specs=[
            pl.BlockSpec((blk, 1), lambda i: (i, 0)),
            pl.BlockSpec((blk, d), lambda i: (i, 0)),
            pl.BlockSpec((1, 1), lambda i: (0, 0)),
        ],
        out_shape=[
            jax.ShapeDtypeStruct((n, 1), jnp.int32),
            jax.ShapeDtypeStruct((n, d), jnp.float32),
            jax.ShapeDtypeStruct((1, 1), jnp.float32),
        ],
    )(flat, codebook, fn, cbn)
    return idx, q, loss[0, 0]


_DNL = ('NHWC', 'HWIO', 'NHWC')


def _conv2d_nhwc(x, w, b, stride, pad):
    y = jax.lax.conv_general_dilated(
        x, w.transpose(2, 3, 1, 0), (stride, stride),
        [(pad, pad), (pad, pad)], dimension_numbers=_DNL)
    return y + b[None, None, None, :]


_TAPS = ((0, 0), (0, 1), (1, 0), (1, 1))


def _dec_body(x_ref, w_ref, b_ref, out_ref, *, gh, gw, valid, shifts,
              store_rows, store_cols, relu_out):
    r_in = x_ref.shape[1]
    ci = x_ref.shape[3]
    xv = x_ref[0].reshape(r_in * gw, ci).astype(jnp.bfloat16)
    m = gh * gw
    b = b_ref[...]
    nr_out, nc_out, co = out_ref.shape[1], out_ref.shape[2], out_ref.shape[3]
    lo = store_rows
    hi = lo + 2 * valid
    out_ref[0, 0:lo, :, :] = jnp.zeros((lo, nc_out, co), out_ref.dtype)
    out_ref[0, hi:, :, :] = jnp.zeros((nr_out - hi, nc_out, co),
                                      out_ref.dtype)
    out_ref[0, :, 0:lo, :] = jnp.zeros((nr_out, lo, co), out_ref.dtype)
    out_ref[0, :, hi:, :] = jnp.zeros((nr_out, nc_out - hi, co),
                                      out_ref.dtype)
    for p, (pr, pc) in enumerate(((0, 0), (0, 1), (1, 0), (1, 1))):
        lhs = jnp.concatenate(
            [xv[s:s + m] for s in shifts[p]], axis=1)
        acc = jax.lax.dot_general(
            lhs, w_ref[p], dimension_numbers=(((1,), (0,)), ((), ())),
            preferred_element_type=jnp.float32)
        y = acc + b
        y = jnp.maximum(y, 0.0) if relu_out else jax.nn.sigmoid(y)
        y = y.astype(out_ref.dtype).reshape(gh, gw, -1)[:valid, :valid, :]
        out_ref[0, pl.Slice(store_rows + pr, valid, 2),
                pl.Slice(store_cols + pc, valid, 2), :] = y


def _dec_layer(xbuf, w, bias, *, gh, gw, valid, taps_kh, shifts,
               out_rows, out_cols, out_co, store_rows, store_cols,
               relu_out):
    B = xbuf.shape[0]
    Ci = xbuf.shape[3]
    Co = w.shape[1]
    wp = jnp.stack([
        jnp.concatenate([w[:, :, kh, kw] for (kh, kw) in taps_kh[p]], axis=0)
        for p in range(4)]).astype(jnp.bfloat16)
    out = pl.pallas_call(
        functools.partial(
            _dec_body, gh=gh, gw=gw, valid=valid, shifts=shifts,
            store_rows=store_rows, store_cols=store_cols, relu_out=relu_out),
        grid=(B,),
        in_specs=[
            pl.BlockSpec((1,) + xbuf.shape[1:], lambda i: (i, 0, 0, 0)),
            pl.BlockSpec((4, 4 * Ci, Co), lambda i: (0, 0, 0)),
            pl.BlockSpec((1, Co), lambda i: (0, 0)),
        ],
        out_specs=pl.BlockSpec((1, out_rows, out_cols, Co),
                               lambda i: (i, 0, 0, 0)),
        out_shape=jax.ShapeDtypeStruct((B, out_rows, out_cols, Co),
                                       jnp.float32),
    )(xbuf, wp, bias.reshape(1, Co).astype(jnp.float32))
    return out


def _dec3_body(x_ref, w_ref, b_ref, out_ref, *, gh, gw, shifts, co_out, hout):
    r_in = x_ref.shape[1]
    ci = x_ref.shape[3]
    xv = x_ref[0].reshape(r_in * gw, ci).astype(jnp.bfloat16)
    m = gh * gw
    lhs = jnp.concatenate([xv[s:s + m] for s in shifts], axis=1)
    acc = jax.lax.dot_general(
        lhs, w_ref[...], dimension_numbers=(((1,), (0,)), ((), ())),
        preferred_element_type=jnp.float32)
    y = jax.nn.sigmoid(acc + b_ref[...]).reshape(gh, gw, 4 * co_out)
    half = (hout + 1) // 2
    for co in range(co_out):
        for r in range(2):
            nr = half if r == 0 else hout - half
            t = y[:nr, :half, 2 * (2 * co + r):2 * (2 * co + r) + 2]
            t = t.reshape(nr, 2 * half)
            wpad = 2 * out_ref.shape[4] - 2 * half
            t = jnp.pad(t, ((0, 0), (0, wpad)))
            out_ref[0, co, pl.Slice(r, nr, 2), :, :] = (
                t.reshape(nr, 2, out_ref.shape[4]))


def kernel(x, ew1, eb1, ew2, eb2, ew3, eb3, codebook, dw1, db1, dw2, db2,
           dw3, db3):
    commitment_cost = 0.25
    xl = x.transpose(0, 2, 3, 1)
    z = jax.nn.relu(_conv2d_nhwc(xl, ew1, eb1, 2, 1))
    z = jax.nn.relu(_conv2d_nhwc(z, ew2, eb2, 2, 1))
    z = _conv2d_nhwc(z, ew3, eb3, 2, 1)
    B, H, W, D = z.shape
    flat = z.reshape(-1, D)

    idx, quantized, loss_sum = _vq(flat, codebook)
    vq_loss = (1.0 + commitment_cost) * loss_sum / (flat.shape[0] * D)

    zq = jnp.pad(quantized.reshape(B, H, W, D),
                 ((0, 0), (1, 5), (1, 3), (0, 0)))
    taps1 = [[(3 - 2 * a - r, 3 - 2 * bb - c) for (a, bb) in _TAPS]
             for r in (0, 1) for c in (0, 1)]
    shifts1 = [tuple((r + a) * 32 + (c + bb) for (a, bb) in _TAPS)
               for r in (0, 1) for c in (0, 1)]
    h1 = _dec_layer(zq, dw1, db1, gh=30, gw=32, valid=28, taps_kh=taps1,
                    shifts=shifts1, out_rows=64, out_cols=64, out_co=64,
                    store_rows=1, store_cols=1, relu_out=True)
    shifts2 = [tuple((r + a) * 64 + (c + bb) for (a, bb) in _TAPS)
               for r in (0, 1) for c in (0, 1)]
    h2 = _dec_layer(h1, dw2, db2, gh=58, gw=64, valid=56, taps_kh=taps1,
                    shifts=shifts2, out_rows=120, out_cols=120, out_co=32,
                    store_rows=1, store_cols=1, relu_out=True)
    Ci3, Co3 = dw3.shape[0], dw3.shape[1]
    w3 = jnp.concatenate([
        jnp.stack([dw3[:, co, r + 2 * a, c + 2 * bb]
                   for co in range(Co3) for r in (0, 1) for c in (0, 1)],
                  axis=1)
        for (a, bb) in _TAPS], axis=0).astype(jnp.bfloat16)
    b3 = jnp.repeat(db3, 4).reshape(1, 4 * Co3).astype(jnp.float32)
    shifts3 = tuple((1 - a) * 120 + (1 - bb) for (a, bb) in _TAPS)
    hout = 2 * 112 + 3
    x_recon = pl.pallas_call(
        functools.partial(_dec3_body, gh=115, gw=120, shifts=shifts3,
                          co_out=Co3, hout=hout),
        grid=(B,),
        in_specs=[
            pl.BlockSpec((1, 120, 120, Ci3), lambda i: (i, 0, 0, 0)),
            pl.BlockSpec((4 * Ci3, 4 * Co3), lambda i: (0, 0)),
            pl.BlockSpec((1, 4 * Co3), lambda i: (0, 0)),
        ],
        out_specs=pl.BlockSpec((1, Co3, hout, 2, 128),
                               lambda i: (i, 0, 0, 0, 0)),
        out_shape=jax.ShapeDtypeStruct((B, Co3, hout, 2, 128), jnp.float32),
    )(h2, w3, b3)
    x_recon = x_recon.reshape(B, Co3, hout, 256)[:, :, :, :hout]
    return (x_recon, vq_loss, idx)
